# baseline (speedup 1.0000x reference)
import functools

import jax
import jax.numpy as jnp
from jax import lax
from jax.experimental import pallas as pl
from jax.experimental.pallas import tpu as pltpu
from jax.experimental.pallas import tpu_sc as plsc

_N = 16384
_D = 32
_NC = 2
_NS = 16
_NW = _NC * _NS
_BPW = _N // _NW
_CHUNK = 128
_NCHUNK = _BPW // _CHUNK
_GRP = 16
_R = 128


def _sc_scores(u2, i2, j2, bu, bi, p, q):
    mesh = plsc.VectorSubcoreMesh(core_axis_name="c", subcore_axis_name="s")

    @functools.partial(
        pl.kernel,
        mesh=mesh,
        compiler_params=pltpu.CompilerParams(
            use_tc_tiling_on_sc=False, needs_layout_passes=False
        ),
        out_type=(
            jax.ShapeDtypeStruct((_R, _R), jnp.float32),
            jax.ShapeDtypeStruct((_R, _R), jnp.float32),
        ),
        scratch_types=[
            pltpu.VMEM((_NCHUNK, _CHUNK), jnp.int32),
            pltpu.VMEM((_NCHUNK, _CHUNK), jnp.int32),
            pltpu.VMEM((_NCHUNK, _CHUNK), jnp.int32),
            pltpu.VMEM((_NCHUNK, _CHUNK, _D), jnp.float32),
            pltpu.VMEM((_NCHUNK, _CHUNK, _D), jnp.float32),
            pltpu.VMEM((_NCHUNK, _CHUNK, _D), jnp.float32),
            pltpu.VMEM((_NCHUNK, _CHUNK), jnp.float32),
            pltpu.VMEM((_NCHUNK, _CHUNK), jnp.float32),
            pltpu.VMEM((_NCHUNK, _CHUNK), jnp.float32),
            pltpu.VMEM((_NCHUNK, _CHUNK), jnp.float32),
            pltpu.VMEM((_NCHUNK, _CHUNK), jnp.float32),
            pltpu.SemaphoreType.DMA,
        ],
    )
    def k(u_hbm, i_hbm, j_hbm, bu_hbm, bi_hbm, p_hbm, q_hbm,
          rui_hbm, ruj_hbm,
          u_v, i_v, j_v, pu_v, qi_v, qj_v, gbu_v, gbi_v, gbj_v,
          rui_v, ruj_v, sem):
        wid = lax.axis_index("s") * _NC + lax.axis_index("c")
        row0 = wid * _NCHUNK

        pltpu.sync_copy(u_hbm.at[pl.ds(row0, _NCHUNK)], u_v)
        pltpu.sync_copy(i_hbm.at[pl.ds(row0, _NCHUNK)], i_v)
        pltpu.sync_copy(j_hbm.at[pl.ds(row0, _NCHUNK)], j_v)

        cps = []
        for c in range(_NCHUNK):
            cps.append(pltpu.async_copy(p_hbm.at[u_v.at[c]], pu_v.at[c], sem))
            cps.append(pltpu.async_copy(q_hbm.at[i_v.at[c]], qi_v.at[c], sem))
            cps.append(pltpu.async_copy(q_hbm.at[j_v.at[c]], qj_v.at[c], sem))
            cps.append(pltpu.async_copy(bu_hbm.at[u_v.at[c]], gbu_v.at[c], sem))
            cps.append(pltpu.async_copy(bi_hbm.at[i_v.at[c]], gbi_v.at[c], sem))
            cps.append(pltpu.async_copy(bi_hbm.at[j_v.at[c]], gbj_v.at[c], sem))
        for cp in cps:
            cp.wait()

        lane = lax.broadcasted_iota(jnp.int32, (_GRP,), 0)
        for c in range(_NCHUNK):
            cvec = jnp.full((_GRP,), c, jnp.int32)

            def body(g, carry, cvec=cvec, c=c):
                row = g * _GRP + lane
                acc_i = jnp.zeros((_GRP,), jnp.float32)
                acc_j = jnp.zeros((_GRP,), jnp.float32)
                for dd in range(_D):
                    col = jnp.full((_GRP,), dd, jnp.int32)
                    pu_d = plsc.load_gather(pu_v, [cvec, row, col])
                    qi_d = plsc.load_gather(qi_v, [cvec, row, col])
                    qj_d = plsc.load_gather(qj_v, [cvec, row, col])
                    acc_i = acc_i + pu_d * qi_d
                    acc_j = acc_j + pu_d * qj_d
                b_u = plsc.load_gather(gbu_v, [cvec, row])
                b_i = plsc.load_gather(gbi_v, [cvec, row])
                b_j = plsc.load_gather(gbj_v, [cvec, row])
                plsc.store_scatter(rui_v, [cvec, row], b_u + b_i + acc_i)
                plsc.store_scatter(ruj_v, [cvec, row], b_u + b_j + acc_j)
                return carry

            lax.fori_loop(0, _CHUNK // _GRP, body, 0)

        pltpu.sync_copy(rui_v, rui_hbm.at[pl.ds(row0, _NCHUNK)])
        pltpu.sync_copy(ruj_v, ruj_hbm.at[pl.ds(row0, _NCHUNK)])

    return k(u2, i2, j2, bu, bi, p, q)


def _loss_body(m_ref, rui_ref, ruj_ref, ui_ref, uj_ref, out_ref):
    m = m_ref[0]
    rui = rui_ref[...] + m
    ruj = ruj_ref[...] + m
    r = rui - ruj
    bpr = jnp.maximum(-r, 0.0) + jnp.log1p(jnp.exp(-jnp.abs(r)))
    d1 = rui - ui_ref[...]
    a1 = jnp.abs(d1)
    s1 = jnp.where(a1 < 1.0, 0.5 * d1 * d1, a1 - 0.5)
    d2 = ruj - uj_ref[...]
    a2 = jnp.abs(d2)
    s2 = jnp.where(a2 < 1.0, 0.5 * d2 * d2, a2 - 0.5)
    out_ref[0, 0] = (0.7 * jnp.mean(bpr)
                     + 0.3 * 0.5 * (jnp.mean(s1) + jnp.mean(s2)))


def kernel(u, i, j, ui, uj, m, bu, bi, p, q):
    u2 = jnp.reshape(u.astype(jnp.int32), (_R, _R))
    i2 = jnp.reshape(i.astype(jnp.int32), (_R, _R))
    j2 = jnp.reshape(j.astype(jnp.int32), (_R, _R))
    rui, ruj = _sc_scores(u2, i2, j2, bu, bi, p, q)
    out = pl.pallas_call(
        _loss_body,
        out_shape=jax.ShapeDtypeStruct((1, 1), jnp.float32),
        in_specs=[
            pl.BlockSpec(memory_space=pltpu.SMEM),
            pl.BlockSpec(memory_space=pltpu.VMEM),
            pl.BlockSpec(memory_space=pltpu.VMEM),
            pl.BlockSpec(memory_space=pltpu.VMEM),
            pl.BlockSpec(memory_space=pltpu.VMEM),
        ],
        out_specs=pl.BlockSpec(memory_space=pltpu.SMEM),
    )(m, rui, ruj, jnp.reshape(ui, (_R, _R)), jnp.reshape(uj, (_R, _R)))
    return out[0, 0]
